# baseline (speedup 1.0000x reference)
import functools

import jax
import jax.numpy as jnp
from jax import lax
from jax.experimental import pallas as pl
from jax.experimental.pallas import tpu as pltpu
from jax.experimental.pallas import tpu_sc as plsc

NUM_CORES = 2
NUM_SUBCORES = 16
NUM_WORKERS = NUM_CORES * NUM_SUBCORES

BATCH = 16384
EMBED_DIM = 128
B_PER_W = BATCH // NUM_WORKERS
CHUNK = 64
NBUF = 8
CHUNKS_PER_TABLE = B_PER_W // CHUNK
NCHUNKS = 2 * CHUNKS_PER_TABLE


def _build_lookup():
    mesh = plsc.VectorSubcoreMesh(core_axis_name="c", subcore_axis_name="s")

    @functools.partial(
        pl.kernel,
        mesh=mesh,
        out_type=[
            jax.ShapeDtypeStruct((BATCH, EMBED_DIM), jnp.float32),
            jax.ShapeDtypeStruct((BATCH, EMBED_DIM), jnp.float32),
        ],
        scratch_types=(
            [pltpu.VMEM((B_PER_W,), jnp.int32)] * 2
            + [pltpu.VMEM((CHUNK, EMBED_DIM), jnp.float32)] * NBUF
            + [pltpu.SemaphoreType.DMA] * (2 * NBUF + 1)
        ),
    )
    def lookup(uidx_hbm, gidx_hbm, utab_hbm, gtab_hbm,
               uout_hbm, gout_hbm, uidx_v, gidx_v, *rest):
        bufs = rest[:NBUF]
        gsems = rest[NBUF:2 * NBUF]
        wsems = rest[2 * NBUF:3 * NBUF]
        isem = rest[3 * NBUF]
        wid = lax.axis_index("s") * NUM_CORES + lax.axis_index("c")
        base = wid * B_PER_W

        idx_cp_u = pltpu.async_copy(uidx_hbm.at[pl.ds(base, B_PER_W)], uidx_v, isem)
        idx_cp_g = pltpu.async_copy(gidx_hbm.at[pl.ds(base, B_PER_W)], gidx_v, isem)
        idx_cp_u.wait()
        idx_cp_g.wait()

        def chunk_refs(i):
            if i < CHUNKS_PER_TABLE:
                off = i * CHUNK
                return uidx_v.at[pl.ds(off, CHUNK)], utab_hbm, uout_hbm, off
            off = (i - CHUNKS_PER_TABLE) * CHUNK
            return gidx_v.at[pl.ds(off, CHUNK)], gtab_hbm, gout_hbm, off

        def start_gather(i, b):
            idx, tab, _, _ = chunk_refs(i)
            return pltpu.async_copy(tab.at[idx], bufs[b], gsems[b])

        def start_write(i, b):
            _, _, out, off = chunk_refs(i)
            return pltpu.async_copy(bufs[b], out.at[pl.ds(base + off, CHUNK)],
                                    wsems[b])

        gcp = [start_gather(b, b) for b in range(NBUF)] + [None] * (NCHUNKS - NBUF)
        wcp = [None] * NCHUNKS
        for i in range(NCHUNKS):
            b = i % NBUF
            gcp[i].wait()
            wcp[i] = start_write(i, b)
            if i + NBUF < NCHUNKS:
                wcp[i].wait()
                gcp[i + NBUF] = start_gather(i + NBUF, b)
        for i in range(NCHUNKS - NBUF, NCHUNKS):
            wcp[i].wait()

    return lookup


_lookup = _build_lookup()


def kernel(user_input, game_input, user_table, game_table):
    out = _lookup(user_input.astype(jnp.int32), game_input.astype(jnp.int32),
                  user_table, game_table)
    return (out[0], out[1])


# speedup vs baseline: 1.0056x; 1.0056x over previous
import functools

import jax
import jax.numpy as jnp
from jax import lax
from jax.experimental import pallas as pl
from jax.experimental.pallas import tpu as pltpu
from jax.experimental.pallas import tpu_sc as plsc

NUM_CORES = 2
NUM_SUBCORES = 16
NUM_WORKERS = NUM_CORES * NUM_SUBCORES

BATCH = 16384
EMBED_DIM = 128
B_PER_W = BATCH // NUM_WORKERS
CHUNK = 128
NBUF = 4
CHUNKS_PER_TABLE = B_PER_W // CHUNK
NCHUNKS = 2 * CHUNKS_PER_TABLE


def _build_lookup():
    mesh = plsc.VectorSubcoreMesh(core_axis_name="c", subcore_axis_name="s")

    @functools.partial(
        pl.kernel,
        mesh=mesh,
        out_type=[
            jax.ShapeDtypeStruct((BATCH, EMBED_DIM), jnp.float32),
            jax.ShapeDtypeStruct((BATCH, EMBED_DIM), jnp.float32),
        ],
        scratch_types=(
            [pltpu.VMEM((B_PER_W,), jnp.int32)] * 2
            + [pltpu.VMEM((CHUNK, EMBED_DIM), jnp.float32)] * NBUF
            + [pltpu.SemaphoreType.DMA] * (2 * NBUF + 1)
        ),
    )
    def lookup(uidx_hbm, gidx_hbm, utab_hbm, gtab_hbm,
               uout_hbm, gout_hbm, uidx_v, gidx_v, *rest):
        bufs = rest[:NBUF]
        gsems = rest[NBUF:2 * NBUF]
        wsems = rest[2 * NBUF:3 * NBUF]
        isem = rest[3 * NBUF]
        wid = lax.axis_index("s") * NUM_CORES + lax.axis_index("c")
        base = wid * B_PER_W

        idx_cp_u = pltpu.async_copy(uidx_hbm.at[pl.ds(base, B_PER_W)], uidx_v, isem)
        idx_cp_g = pltpu.async_copy(gidx_hbm.at[pl.ds(base, B_PER_W)], gidx_v, isem)
        idx_cp_u.wait()
        idx_cp_g.wait()

        def chunk_refs(i):
            if i < CHUNKS_PER_TABLE:
                off = i * CHUNK
                return uidx_v.at[pl.ds(off, CHUNK)], utab_hbm, uout_hbm, off
            off = (i - CHUNKS_PER_TABLE) * CHUNK
            return gidx_v.at[pl.ds(off, CHUNK)], gtab_hbm, gout_hbm, off

        def start_gather(i, b):
            idx, tab, _, _ = chunk_refs(i)
            return pltpu.async_copy(tab.at[idx], bufs[b], gsems[b])

        def start_write(i, b):
            _, _, out, off = chunk_refs(i)
            return pltpu.async_copy(bufs[b], out.at[pl.ds(base + off, CHUNK)],
                                    wsems[b])

        gcp = [start_gather(b, b) for b in range(NBUF)] + [None] * (NCHUNKS - NBUF)
        wcp = [None] * NCHUNKS
        for i in range(NCHUNKS):
            b = i % NBUF
            gcp[i].wait()
            wcp[i] = start_write(i, b)
            if i + NBUF < NCHUNKS:
                wcp[i].wait()
                gcp[i + NBUF] = start_gather(i + NBUF, b)
        for i in range(NCHUNKS - NBUF, NCHUNKS):
            wcp[i].wait()

    return lookup


_lookup = _build_lookup()


def kernel(user_input, game_input, user_table, game_table):
    out = _lookup(user_input.astype(jnp.int32), game_input.astype(jnp.int32),
                  user_table, game_table)
    return (out[0], out[1])
